# baseline (speedup 1.0000x reference)
import functools
import jax
import jax.numpy as jnp
from jax import lax
from jax.experimental import pallas as pl
from jax.experimental.pallas import tpu as pltpu
from jax.experimental.pallas import tpu_sc as plsc

NC = 1000
N = 4096 * 26
NWORK = 32
RPW = N // NWORK
CH = 32
NCHUNK = RPW // CH
L = 16
CHW = CH * NC


def _sc_body(x_hbm, out_hbm, idx_v, buf0, buf1, sem0, sem1):
    wid = lax.axis_index("s") * 2 + lax.axis_index("c")
    base = wid * RPW

    pltpu.sync_copy(x_hbm.at[pl.ds(base, RPW)], idx_v)

    zeros = jnp.zeros((L,), jnp.float32)
    ones = jnp.ones((L,), jnp.float32)
    lanes = lax.iota(jnp.int32, L)

    def _zero(i, _):
        buf0[pl.ds(i * L, L)] = zeros
        buf1[pl.ds(i * L, L)] = zeros
        return 0

    lax.fori_loop(0, CHW // L, _zero, 0)

    def _positions(c, j):
        v = idx_v[pl.ds(c * CH + j * L, L)]
        return (j * L + lanes) * NC + v

    def _scatter(buf, c, val):
        for j in range(CH // L):
            plsc.store_scatter(buf, [_positions(c, j)], val)

    def _dma(buf, sem, c):
        return pltpu.make_async_copy(
            buf, out_hbm.at[pl.ds((base + c * CH) * NC, CHW)], sem)

    _scatter(buf0, 0, ones)
    _dma(buf0, sem0, 0).start()
    _scatter(buf1, 1, ones)
    _dma(buf1, sem1, 1).start()

    def _pair(i, _):
        c0 = 2 * i
        _dma(buf0, sem0, c0 - 2).wait()
        _scatter(buf0, c0 - 2, zeros)
        _scatter(buf0, c0, ones)
        _dma(buf0, sem0, c0).start()
        c1 = 2 * i + 1
        _dma(buf1, sem1, c1 - 2).wait()
        _scatter(buf1, c1 - 2, zeros)
        _scatter(buf1, c1, ones)
        _dma(buf1, sem1, c1).start()
        return 0

    lax.fori_loop(1, NCHUNK // 2, _pair, 0)

    _dma(buf0, sem0, NCHUNK - 2).wait()
    _dma(buf1, sem1, NCHUNK - 1).wait()


@jax.jit
def kernel(x):
    mesh = plsc.VectorSubcoreMesh(core_axis_name="c", subcore_axis_name="s")
    run = pl.kernel(
        _sc_body,
        mesh=mesh,
        compiler_params=pltpu.CompilerParams(needs_layout_passes=False),
        out_type=jax.ShapeDtypeStruct((N * NC,), jnp.float32),
        scratch_types=[
            pltpu.VMEM((RPW,), jnp.int32),
            pltpu.VMEM((CHW,), jnp.float32),
            pltpu.VMEM((CHW,), jnp.float32),
            pltpu.SemaphoreType.DMA,
            pltpu.SemaphoreType.DMA,
        ],
    )
    out = run(x.reshape(-1).astype(jnp.int32))
    return out.reshape(4096, 26, NC)


# speedup vs baseline: 1.9600x; 1.9600x over previous
import functools
import jax
import jax.numpy as jnp
from jax import lax
from jax.experimental import pallas as pl
from jax.experimental.pallas import tpu as pltpu
from jax.experimental.pallas import tpu_sc as plsc

B = 4096
C = 26
NC = 1000
NWORK = 32
BPW = B // NWORK
CHB = 1
NCHUNK = BPW // CHB
FR = CHB * C
L = 16


def _sc_body(x_hbm, out_hbm, idx_v, buf0, buf1, sem0, sem1):
    wid = lax.axis_index("s") * 2 + lax.axis_index("c")
    base = wid * BPW

    pltpu.sync_copy(x_hbm.at[pl.ds(base * C, BPW * C)], idx_v)

    zeros = jnp.zeros((L,), jnp.float32)
    ones = jnp.ones((L,), jnp.float32)
    lanes = lax.iota(jnp.int32, L)

    def _zero_row(buf, r, c):
        for k in range(NC // L):
            buf[r, c, pl.ds(k * L, L)] = zeros
        buf[r, c, pl.ds(NC - L, L)] = zeros

    for r in range(CHB):
        for c in range(C):
            _zero_row(buf0, r, c)
            _zero_row(buf1, r, c)

    row0 = jnp.zeros((L,), jnp.int32)

    def _scatter(buf, n, val):
        for j in range(2):
            off = j * (FR - L)
            c = off + lanes
            v = idx_v[pl.ds(n * FR + off, L)]
            plsc.store_scatter(buf, [row0, c, v], val)

    def _dma(buf, sem, n):
        return pltpu.make_async_copy(
            buf, out_hbm.at[pl.ds(base + n * CHB, CHB)], sem)

    _scatter(buf0, 0, ones)
    _dma(buf0, sem0, 0).start()
    _scatter(buf1, 1, ones)
    _dma(buf1, sem1, 1).start()

    def _pair(i, _):
        n0 = 2 * i
        _dma(buf0, sem0, n0 - 2).wait()
        _scatter(buf0, n0 - 2, zeros)
        _scatter(buf0, n0, ones)
        _dma(buf0, sem0, n0).start()
        n1 = 2 * i + 1
        _dma(buf1, sem1, n1 - 2).wait()
        _scatter(buf1, n1 - 2, zeros)
        _scatter(buf1, n1, ones)
        _dma(buf1, sem1, n1).start()
        return 0

    lax.fori_loop(1, NCHUNK // 2, _pair, 0)

    _dma(buf0, sem0, NCHUNK - 2).wait()
    _dma(buf1, sem1, NCHUNK - 1).wait()


@jax.jit
def kernel(x):
    mesh = plsc.VectorSubcoreMesh(core_axis_name="c", subcore_axis_name="s")
    run = pl.kernel(
        _sc_body,
        mesh=mesh,
        compiler_params=pltpu.CompilerParams(needs_layout_passes=False),
        out_type=jax.ShapeDtypeStruct((B, C, NC), jnp.float32),
        scratch_types=[
            pltpu.VMEM((BPW * C,), jnp.int32),
            pltpu.VMEM((CHB, C, NC), jnp.float32),
            pltpu.VMEM((CHB, C, NC), jnp.float32),
            pltpu.SemaphoreType.DMA,
            pltpu.SemaphoreType.DMA,
        ],
    )
    return run(x.reshape(-1).astype(jnp.int32))


# speedup vs baseline: 8.2357x; 4.2018x over previous
import functools
import jax
import jax.numpy as jnp
from jax import lax
from jax.experimental import pallas as pl
from jax.experimental.pallas import tpu as pltpu
from jax.experimental.pallas import tpu_sc as plsc

B = 4096
C = 26
NC = 1000
NWORK = 32
BPW = B // NWORK
KC = 200
KCH = NC // KC
NBLK = C * KCH
L = 16


def _sc_body(xt_hbm, out_hbm, idx_v, buf0, buf1, sem0, sem1):
    wid = lax.axis_index("s") * 2 + lax.axis_index("c")
    b0 = wid * BPW

    pltpu.sync_copy(xt_hbm.at[:, pl.ds(b0, BPW)], idx_v)

    zeros = jnp.zeros((L,), jnp.float32)
    ones = jnp.ones((L,), jnp.float32)
    lanes = lax.iota(jnp.int32, L)

    def _zero(r, _):
        for j in range(BPW // L):
            buf0[r, pl.ds(j * L, L)] = zeros
            buf1[r, pl.ds(j * L, L)] = zeros
        return 0

    lax.fori_loop(0, KC, _zero, 0)

    def _scatter(buf, n, val):
        c = n // KCH
        k0 = (n % KCH) * KC
        for j in range(BPW // L):
            v = idx_v[c, pl.ds(j * L, L)]
            kk = v - k0
            msk = (kk >= 0) & (kk < KC)
            plsc.store_scatter(buf, [kk, j * L + lanes], val, mask=msk)

    def _dma(buf, sem, n):
        c = n // KCH
        k0 = (n % KCH) * KC
        return pltpu.make_async_copy(
            buf, out_hbm.at[c, pl.ds(k0, KC), pl.ds(b0, BPW)], sem)

    _scatter(buf0, 0, ones)
    _dma(buf0, sem0, 0).start()
    _scatter(buf1, 1, ones)
    _dma(buf1, sem1, 1).start()

    def _pair(i, _):
        n0 = 2 * i
        _dma(buf0, sem0, n0 - 2).wait()
        _scatter(buf0, n0 - 2, zeros)
        _scatter(buf0, n0, ones)
        _dma(buf0, sem0, n0).start()
        n1 = 2 * i + 1
        _dma(buf1, sem1, n1 - 2).wait()
        _scatter(buf1, n1 - 2, zeros)
        _scatter(buf1, n1, ones)
        _dma(buf1, sem1, n1).start()
        return 0

    lax.fori_loop(1, NBLK // 2, _pair, 0)

    _dma(buf0, sem0, NBLK - 2).wait()
    _dma(buf1, sem1, NBLK - 1).wait()


@jax.jit
def kernel(x):
    mesh = plsc.VectorSubcoreMesh(core_axis_name="c", subcore_axis_name="s")
    run = pl.kernel(
        _sc_body,
        mesh=mesh,
        compiler_params=pltpu.CompilerParams(needs_layout_passes=False),
        out_type=jax.ShapeDtypeStruct((C, NC, B), jnp.float32),
        scratch_types=[
            pltpu.VMEM((C, BPW), jnp.int32),
            pltpu.VMEM((KC, BPW), jnp.float32),
            pltpu.VMEM((KC, BPW), jnp.float32),
            pltpu.SemaphoreType.DMA,
            pltpu.SemaphoreType.DMA,
        ],
    )
    out = run(x.T.astype(jnp.int32))
    return out.transpose(2, 0, 1)


# speedup vs baseline: 8.2902x; 1.0066x over previous
import functools
import jax
import jax.numpy as jnp
from jax import lax
from jax.experimental import pallas as pl
from jax.experimental.pallas import tpu as pltpu
from jax.experimental.pallas import tpu_sc as plsc

B = 4096
C = 26
NC = 1000
NWORK = 32
BPW = B // NWORK
KC = 200
KCH = NC // KC
NBLK = C * KCH
L = 16


def _sc_body(xt_hbm, out_hbm, idx_v, buf0, buf1, sem0, sem1):
    wid = lax.axis_index("s") * 2 + lax.axis_index("c")
    b0 = wid * BPW

    pltpu.sync_copy(xt_hbm.at[:, pl.ds(b0, BPW)], idx_v)

    zeros = jnp.zeros((L,), jnp.float32)
    ones = jnp.ones((L,), jnp.float32)
    lanes = lax.iota(jnp.int32, L)

    def _zero(buf):
        def body(r, _):
            for j in range(BPW // L):
                buf[r, pl.ds(j * L, L)] = zeros
            return 0
        lax.fori_loop(0, KC, body, 0)

    def _scatter(buf, n, val):
        c = n // KCH
        k0 = (n % KCH) * KC
        for j in range(BPW // L):
            v = idx_v[c, pl.ds(j * L, L)]
            kk = v - k0
            msk = (kk >= 0) & (kk < KC)
            plsc.store_scatter(buf, [kk, j * L + lanes], val, mask=msk)

    def _dma(buf, sem, n):
        c = n // KCH
        k0 = (n % KCH) * KC
        return pltpu.make_async_copy(
            buf, out_hbm.at[c, pl.ds(k0, KC), pl.ds(b0, BPW)], sem)

    _zero(buf0)
    _scatter(buf0, 0, ones)
    _dma(buf0, sem0, 0).start()
    _zero(buf1)
    _scatter(buf1, 1, ones)
    _dma(buf1, sem1, 1).start()

    def _pair(i, _):
        n0 = 2 * i
        _dma(buf0, sem0, n0 - 2).wait()
        _scatter(buf0, n0 - 2, zeros)
        _scatter(buf0, n0, ones)
        _dma(buf0, sem0, n0).start()
        n1 = 2 * i + 1
        _dma(buf1, sem1, n1 - 2).wait()
        _scatter(buf1, n1 - 2, zeros)
        _scatter(buf1, n1, ones)
        _dma(buf1, sem1, n1).start()
        return 0

    lax.fori_loop(1, NBLK // 2, _pair, 0)

    _dma(buf0, sem0, NBLK - 2).wait()
    _dma(buf1, sem1, NBLK - 1).wait()


@jax.jit
def kernel(x):
    mesh = plsc.VectorSubcoreMesh(core_axis_name="c", subcore_axis_name="s")
    run = pl.kernel(
        _sc_body,
        mesh=mesh,
        compiler_params=pltpu.CompilerParams(needs_layout_passes=False),
        out_type=jax.ShapeDtypeStruct((C, NC, B), jnp.float32),
        scratch_types=[
            pltpu.VMEM((C, BPW), jnp.int32),
            pltpu.VMEM((KC, BPW), jnp.float32),
            pltpu.VMEM((KC, BPW), jnp.float32),
            pltpu.SemaphoreType.DMA,
            pltpu.SemaphoreType.DMA,
        ],
    )
    out = run(x.T.astype(jnp.int32))
    return out.transpose(2, 0, 1)


# speedup vs baseline: 9.8540x; 1.1886x over previous
import functools
import jax
import jax.numpy as jnp
from jax import lax
from jax.experimental import pallas as pl
from jax.experimental.pallas import tpu as pltpu

B = 4096
C = 26
NC = 1000
KC = 200
KCH = NC // KC
NBLK = C * KCH


def _tc_body(xt_hbm, out_hbm):
    pl.run_scoped(
        functools.partial(_tc_inner, xt_hbm, out_hbm),
        pltpu.VMEM((C, B), jnp.int32),
        pltpu.VMEM((KC, B), jnp.float32),
        pltpu.VMEM((KC, B), jnp.float32),
        pltpu.SemaphoreType.DMA,
        pltpu.SemaphoreType.DMA,
    )


def _tc_inner(xt_hbm, out_hbm, tc_idx, tbuf0, tbuf1, tsem0, tsem1):
    pltpu.sync_copy(xt_hbm, tc_idx)

    def _fill(buf, n):
        c = n // KCH
        k0 = (n % KCH) * KC
        row = tc_idx[c, :]
        kv = lax.broadcasted_iota(jnp.int32, (KC, B), 0) + k0
        buf[...] = (kv == row[None, :]).astype(jnp.float32)

    def _dma(buf, sem, n):
        c = n // KCH
        k0 = (n % KCH) * KC
        return pltpu.make_async_copy(
            buf, out_hbm.at[c, pl.ds(k0, KC)], sem)

    _fill(tbuf0, 0)
    _dma(tbuf0, tsem0, 0).start()
    _fill(tbuf1, 1)
    _dma(tbuf1, tsem1, 1).start()

    def _pair(i, _):
        n0 = 2 * i
        _dma(tbuf0, tsem0, n0 - 2).wait()
        _fill(tbuf0, n0)
        _dma(tbuf0, tsem0, n0).start()
        n1 = 2 * i + 1
        _dma(tbuf1, tsem1, n1 - 2).wait()
        _fill(tbuf1, n1)
        _dma(tbuf1, tsem1, n1).start()
        return 0

    lax.fori_loop(1, NBLK // 2, _pair, 0)

    _dma(tbuf0, tsem0, NBLK - 2).wait()
    _dma(tbuf1, tsem1, NBLK - 1).wait()


@jax.jit
def kernel(x):
    mesh = pltpu.create_tensorcore_mesh("t")
    run = pl.kernel(
        _tc_body,
        mesh=mesh,
        out_type=jax.ShapeDtypeStruct((C, NC, B), jnp.float32),
    )
    out = run(x.T.astype(jnp.int32))
    return out.transpose(2, 0, 1)
